# baseline (speedup 1.0000x reference)
import functools

import jax
import jax.numpy as jnp
from jax import lax
from jax.experimental import pallas as pl
from jax.experimental.pallas import tpu as pltpu
from jax.experimental.pallas import tpu_sc as plsc

N_ROWS = 10000
B = 4096
D = 64 * 11 * 11
CD = 4
CDP = 16

NC = 2
NS = 16
NW = NC * NS
BPW = B // NW
CHUNK = 8
NCH = BPW // CHUNK


def _sc_gather(slices_hbm, idx_hbm, coords_hbm, out_hbm, lab_hbm,
               idx_v, buf0, buf1, cbuf, sem0, sem1, csem):
    wid = lax.axis_index("s") * NC + lax.axis_index("c")
    base = wid * BPW

    pltpu.sync_copy(idx_hbm.at[pl.ds(base, BPW)], idx_v)

    bufs = (buf0, buf1)
    sems = (sem0, sem1)

    def gather(g):
        return pltpu.async_copy(
            slices_hbm.at[idx_v.at[pl.ds(g * CHUNK, CHUNK)]],
            bufs[g % 2], sems[g % 2])

    h = [None, None]
    h[0] = gather(0)

    ch = pltpu.async_copy(coords_hbm.at[idx_v], cbuf, csem)

    for g in range(NCH):
        nxt = g + 1
        if nxt < NCH:
            h[nxt % 2] = gather(nxt)
        h[g % 2].wait()
        pltpu.sync_copy(bufs[g % 2], out_hbm.at[pl.ds(base + g * CHUNK, CHUNK)])

    ch.wait()
    pltpu.sync_copy(cbuf, lab_hbm.at[pl.ds(base, BPW)])


@jax.jit
def _run(slices2d, indices, coords):
    mesh = plsc.VectorSubcoreMesh(core_axis_name="c", subcore_axis_name="s")
    k = pl.kernel(
        _sc_gather,
        out_type=(
            jax.ShapeDtypeStruct((B, D), jnp.float32),
            jax.ShapeDtypeStruct((B, CDP), jnp.float32),
        ),
        mesh=mesh,
        scratch_types=[
            pltpu.VMEM((BPW,), jnp.int32),
            pltpu.VMEM((CHUNK, D), jnp.float32),
            pltpu.VMEM((CHUNK, D), jnp.float32),
            pltpu.VMEM((BPW, CDP), jnp.float32),
            pltpu.SemaphoreType.DMA,
            pltpu.SemaphoreType.DMA,
            pltpu.SemaphoreType.DMA,
        ],
        compiler_params=pltpu.CompilerParams(use_tc_tiling_on_sc=False),
    )
    return k(slices2d, indices, coords)


def kernel(num_samples, slices, coords):
    key = jax.random.key(1)
    n = slices.shape[0]
    perm = jax.random.permutation(key, n)
    indices = lax.dynamic_slice_in_dim(perm, num_samples - B, B).astype(jnp.int32)
    slices2d = slices.reshape(n, D)
    coords_p = jnp.pad(coords, ((0, 0), (0, CDP - CD)))
    samples2d, labels_p = _run(slices2d, indices, coords_p)
    return (samples2d.reshape(B, *slices.shape[1:]), labels_p[:, :CD])


# speedup vs baseline: 5.1170x; 5.1170x over previous
import jax
import jax.numpy as jnp
from jax import lax
from jax.experimental import pallas as pl
from jax.experimental.pallas import tpu as pltpu
from jax.experimental.pallas import tpu_sc as plsc

N_ROWS = 10000
B = 4096
HW = 121
F = 64
CD = 4
CDP = 16

NC = 2
NS = 16
NW = NC * NS
CPW = B // NW
FB = 8
WORK = HW * (F // FB)
UPW = -(-WORK // NW)
JBLK = 16


def _sc_coords(coords_hbm, idx_hbm, lab_hbm, idx_v, cbuf, csem):
    wid = lax.axis_index("s") * NC + lax.axis_index("c")
    pltpu.sync_copy(idx_hbm.at[pl.ds(wid * CPW, CPW)], idx_v)
    pltpu.async_copy(coords_hbm.at[idx_v], cbuf, csem).wait()
    pltpu.sync_copy(cbuf, lab_hbm.at[pl.ds(wid * CPW, CPW)])


def _sc_gather(tab_hbm, idx_hbm, out_hbm, idx_v, strip, ostrip):
    wid = lax.axis_index("s") * NC + lax.axis_index("c")

    pltpu.sync_copy(idx_hbm, idx_v)

    def unit(t, carry):
        u = wid + t * NW

        @pl.when(u < WORK)
        def _():
            c = u // FB
            fb = u % FB
            pltpu.sync_copy(tab_hbm.at[c, pl.ds(fb * FB, FB)], strip)

            def jblock(jb, carry2):
                cols = idx_v[pl.ds(jb * JBLK, JBLK)]
                for f in range(FB):
                    rowv = jnp.full((JBLK,), f, jnp.int32)
                    v = plsc.load_gather(strip, [rowv, cols])
                    ostrip[f, pl.ds(jb * JBLK, JBLK)] = v
                return carry2

            lax.fori_loop(0, B // JBLK, jblock, 0)
            pltpu.sync_copy(ostrip, out_hbm.at[c, pl.ds(fb * FB, FB)])

        return carry

    lax.fori_loop(0, UPW, unit, 0)


@jax.jit
def _run(tab3, indices, coords_p):
    mesh = plsc.VectorSubcoreMesh(core_axis_name="c", subcore_axis_name="s")
    k = pl.kernel(
        _sc_gather,
        out_type=jax.ShapeDtypeStruct((HW, F, B), jnp.float32),
        mesh=mesh,
        scratch_types=[
            pltpu.VMEM((B,), jnp.int32),
            pltpu.VMEM((FB, N_ROWS), jnp.float32),
            pltpu.VMEM((FB, B), jnp.float32),
        ],
        compiler_params=pltpu.CompilerParams(
            use_tc_tiling_on_sc=True, needs_layout_passes=False),
    )
    kc = pl.kernel(
        _sc_coords,
        out_type=jax.ShapeDtypeStruct((B, CDP), jnp.float32),
        mesh=plsc.VectorSubcoreMesh(core_axis_name="c", subcore_axis_name="s"),
        scratch_types=[
            pltpu.VMEM((CPW,), jnp.int32),
            pltpu.VMEM((CPW, CDP), jnp.float32),
            pltpu.SemaphoreType.DMA,
        ],
        compiler_params=pltpu.CompilerParams(use_tc_tiling_on_sc=False),
    )
    return k(tab3, indices), kc(coords_p, indices)


def kernel(num_samples, slices, coords):
    key = jax.random.key(1)
    n = slices.shape[0]
    perm = jax.random.permutation(key, n)
    indices = lax.dynamic_slice_in_dim(perm, num_samples - B, B).astype(jnp.int32)
    tab3 = jnp.transpose(slices, (2, 3, 1, 0)).reshape(HW, F, n)
    coords_p = jnp.pad(coords, ((0, 0), (0, CDP - CD)))
    out3, labels_p = _run(tab3, indices, coords_p)
    samples = jnp.transpose(
        out3.reshape(11, 11, F, B), (3, 2, 0, 1))
    return (samples, labels_p[:, :CD])
